# baseline (speedup 1.0000x reference)
import functools

import jax
import jax.numpy as jnp
from jax.experimental import pallas as pl
from jax.experimental.pallas import tpu as pltpu

N = 4096
H = 128
T = 50
BM = 512


def _body(aidx_ref, treat_ref, z_ref, w_ref, b_ref, adj_l_ref, adj_r_ref,
          out_ref, xw_ref):
    @pl.when(pl.program_id(0) == 0)
    def _compute_xw():
        row_ids = jax.lax.broadcasted_iota(jnp.int32, (T, 1), 0)
        sel = (row_ids == aidx_ref[0]).astype(jnp.float32)
        m = sel * w_ref[H:H + 1, :]
        zw = jnp.dot(z_ref[...], w_ref[:H, :], preferred_element_type=jnp.float32)
        xw_ref[...] = zw + jnp.dot(treat_ref[...], m,
                                   preferred_element_type=jnp.float32)

    acc = jnp.dot(adj_l_ref[...], xw_ref[:N // 2, :],
                  preferred_element_type=jnp.float32)
    acc += jnp.dot(adj_r_ref[...], xw_ref[N // 2:, :],
                   preferred_element_type=jnp.float32)
    out_ref[...] = jnp.maximum(acc + b_ref[...], 0.0)


@jax.jit
def kernel(t, z, treatments, adj, W, b):
    a_idx = jnp.clip((t * (T - 1)).astype(jnp.int32), 0, T - 1)
    treat2d = treatments[:, :, 0]
    b2d = b.reshape(1, H)

    grid = (N // BM,)
    out = pl.pallas_call(
        _body,
        grid_spec=pltpu.PrefetchScalarGridSpec(
            num_scalar_prefetch=1,
            grid=grid,
            in_specs=[
                pl.BlockSpec((N, T), lambda i, s: (0, 0)),
                pl.BlockSpec((N, H), lambda i, s: (0, 0)),
                pl.BlockSpec((H + 1, H), lambda i, s: (0, 0)),
                pl.BlockSpec((1, H), lambda i, s: (0, 0)),
                pl.BlockSpec((BM, N // 2), lambda i, s: (i, 0)),
                pl.BlockSpec((BM, N // 2), lambda i, s: (i, 1)),
            ],
            out_specs=pl.BlockSpec((BM, H), lambda i, s: (i, 0)),
            scratch_shapes=[pltpu.VMEM((N, H), jnp.float32)],
        ),
        out_shape=jax.ShapeDtypeStruct((N, H), jnp.float32),
        compiler_params=pltpu.CompilerParams(
            dimension_semantics=("arbitrary",),
        ),
    )(a_idx.reshape(1), treat2d, z, W, b2d, adj, adj)
    return out


# speedup vs baseline: 1.0003x; 1.0003x over previous
import jax
import jax.numpy as jnp
from jax.experimental import pallas as pl
from jax.experimental.pallas import tpu as pltpu

N = 4096
H = 128
T = 50
BM = 256
NBUF = 4
NOBUF = 2
G = N // BM


def _body(aidx_ref, treat_ref, z_ref, w_ref, b_ref, adj_hbm, out_hbm,
          xw_ref, *scratch):
    bufs = scratch[:NBUF]
    sems = scratch[NBUF:2 * NBUF]
    obufs = scratch[2 * NBUF:2 * NBUF + NOBUF]
    osems = scratch[2 * NBUF + NOBUF:]

    def adj_copy(i):
        slot = i % NBUF
        return pltpu.make_async_copy(
            adj_hbm.at[pl.ds(i * BM, BM), :], bufs[slot], sems[slot])

    def out_copy(i):
        slot = i % NOBUF
        return pltpu.make_async_copy(
            obufs[slot], out_hbm.at[pl.ds(i * BM, BM), :], osems[slot])

    for k in range(NBUF):
        adj_copy(k).start()

    row_ids = jax.lax.broadcasted_iota(jnp.int32, (T, 1), 0)
    sel = (row_ids == aidx_ref[0]).astype(jnp.float32)
    m = sel * w_ref[H:H + 1, :]
    zw = jnp.dot(z_ref[...], w_ref[:H, :], preferred_element_type=jnp.float32)
    xw_ref[...] = zw + jnp.dot(treat_ref[...], m,
                               preferred_element_type=jnp.float32)

    for i in range(G):
        adj_copy(i).wait()
        if i >= NOBUF:
            out_copy(i - NOBUF).wait()
        acc = jnp.dot(bufs[i % NBUF][...], xw_ref[...],
                      preferred_element_type=jnp.float32)
        obufs[i % NOBUF][...] = jnp.maximum(acc + b_ref[...], 0.0)
        out_copy(i).start()
        if i + NBUF < G:
            adj_copy(i + NBUF).start()

    for i in range(G - NOBUF, G):
        out_copy(i).wait()


@jax.jit
def kernel(t, z, treatments, adj, W, b):
    a_idx = jnp.clip((t * (T - 1)).astype(jnp.int32), 0, T - 1)
    treat2d = treatments[:, :, 0]
    b2d = b.reshape(1, H)

    out = pl.pallas_call(
        _body,
        grid_spec=pltpu.PrefetchScalarGridSpec(
            num_scalar_prefetch=1,
            grid=(1,),
            in_specs=[
                pl.BlockSpec((N, T), lambda i, s: (0, 0)),
                pl.BlockSpec((N, H), lambda i, s: (0, 0)),
                pl.BlockSpec((H + 1, H), lambda i, s: (0, 0)),
                pl.BlockSpec((1, H), lambda i, s: (0, 0)),
                pl.BlockSpec(memory_space=pltpu.MemorySpace.HBM),
            ],
            out_specs=pl.BlockSpec(memory_space=pltpu.MemorySpace.HBM),
            scratch_shapes=(
                [pltpu.VMEM((N, H), jnp.float32)]
                + [pltpu.VMEM((BM, N), jnp.float32) for _ in range(NBUF)]
                + [pltpu.SemaphoreType.DMA for _ in range(NBUF)]
                + [pltpu.VMEM((BM, H), jnp.float32) for _ in range(NOBUF)]
                + [pltpu.SemaphoreType.DMA for _ in range(NOBUF)]
            ),
        ),
        out_shape=jax.ShapeDtypeStruct((N, H), jnp.float32),
        compiler_params=pltpu.CompilerParams(
            dimension_semantics=("arbitrary",),
        ),
    )(a_idx.reshape(1), treat2d, z, W, b2d, adj)
    return out
